# Initial kernel scaffold; baseline (speedup 1.0000x reference)
import functools

import jax
import jax.numpy as jnp
from jax import lax
from jax.experimental import pallas as pl
from jax.experimental.pallas import tpu as pltpu
from jax.experimental.pallas import tpu_sc as plsc

N = 10000
D = 128
E = 320000

NC = 2
NS = 16
NW = NC * NS
CHUNK = 128
EPT = 10112
NCHUNK = EPT // CHUNK
ACC_ROWS = 10016
ZROWS = ACC_ROWS // NS

_mesh = plsc.VectorSubcoreMesh(core_axis_name="c", subcore_axis_name="s")


@functools.partial(
    pl.kernel,
    out_type=jax.ShapeDtypeStruct((NC * ACC_ROWS,), jnp.float32),
    mesh=_mesh,
    scratch_types=[
        pltpu.VMEM((NCHUNK, CHUNK), jnp.int32),
        pltpu.VMEM((CHUNK,), jnp.float32),
        pltpu.VMEM((ZROWS,), jnp.float32),
        pltpu.VMEM_SHARED((ACC_ROWS,), jnp.float32),
    ],
)
def _sc_degree(dst_hbm, ones_hbm, zeros_hbm, out_hbm, dst_v, ones_v, zeros_v,
               hist):
    cid = lax.axis_index("c")
    sid = lax.axis_index("s")
    wid = cid * NS + sid
    pltpu.sync_copy(dst_hbm.at[wid], dst_v)
    pltpu.sync_copy(ones_hbm, ones_v)
    pltpu.sync_copy(zeros_hbm, zeros_v)
    pltpu.sync_copy(zeros_v, hist.at[pl.ds(sid * ZROWS, ZROWS)])
    plsc.subcore_barrier()

    def body(j, carry):
        pltpu.sync_copy(ones_v, hist.at[dst_v.at[j]], add=True)
        return carry

    lax.fori_loop(0, NCHUNK, body, 0)
    plsc.subcore_barrier()
    pltpu.sync_copy(hist.at[pl.ds(sid * ZROWS, ZROWS)],
                    out_hbm.at[pl.ds(cid * ACC_ROWS + sid * ZROWS, ZROWS)])


@functools.partial(
    pl.kernel,
    out_type=jax.ShapeDtypeStruct((NC * ACC_ROWS, D), jnp.float32),
    mesh=_mesh,
    scratch_types=[
        pltpu.VMEM((NCHUNK, CHUNK), jnp.int32),
        pltpu.VMEM((NCHUNK, CHUNK), jnp.int32),
        pltpu.VMEM((2, CHUNK, D), jnp.float32),
        pltpu.VMEM((ZROWS, D), jnp.float32),
        pltpu.VMEM_SHARED((ACC_ROWS, D), jnp.float32),
        pltpu.SemaphoreType.DMA,
        pltpu.SemaphoreType.DMA,
    ],
)
def _sc_scatter(y_hbm, src_hbm, dst_hbm, zeros_hbm, out_hbm, src_v, dst_v,
                buf, zeros_v, acc, sem0, sem1):
    cid = lax.axis_index("c")
    sid = lax.axis_index("s")
    wid = cid * NS + sid
    pltpu.sync_copy(src_hbm.at[wid], src_v)
    pltpu.sync_copy(dst_hbm.at[wid], dst_v)
    pltpu.sync_copy(zeros_hbm, zeros_v)
    pltpu.sync_copy(zeros_v, acc.at[pl.ds(sid * ZROWS, ZROWS)])
    plsc.subcore_barrier()

    sems = (sem0, sem1)
    pltpu.async_copy(y_hbm.at[src_v.at[0]], buf.at[0], sem0)

    def body(j, carry):
        cur = j % 2
        nxt = (j + 1) % 2

        @pl.when(j + 1 < NCHUNK)
        def _():
            pltpu.dma_start(y_hbm.at[src_v.at[j + 1]], buf.at[nxt],
                            sems[0] if False else sem0)

        return carry

    def body2(j, carry):
        @pl.when(j + 1 < NCHUNK)
        def _():
            pltpu.async_copy(y_hbm.at[src_v.at[j + 1]], buf.at[(j + 1) % 2],
                             sem0)

        pltpu.make_async_copy(y_hbm.at[src_v.at[j]], buf.at[j % 2],
                              sem0).wait()
        pltpu.sync_copy(buf.at[j % 2], acc.at[dst_v.at[j]], add=True)
        return carry

    lax.fori_loop(0, NCHUNK, body2, 0)
    plsc.subcore_barrier()
    pltpu.sync_copy(acc.at[pl.ds(sid * ZROWS, ZROWS)],
                    out_hbm.at[pl.ds(cid * ACC_ROWS + sid * ZROWS, ZROWS)])


BR = 1000


def _tc_first_body(h0_ref, h1_ref, x_ref, w_ref, y_ref, dinv_ref):
    deg = h0_ref[...] + h1_ref[...] + 1.0
    dinv = lax.rsqrt(deg)
    xw = jnp.dot(x_ref[...], w_ref[...], preferred_element_type=jnp.float32)
    y_ref[...] = xw * dinv
    dinv_ref[...] = dinv


def _tc_mid_body(a0_ref, a1_ref, y_ref, dinv_ref, b_ref, w_ref, y2_ref):
    dinv = dinv_ref[...]
    h = (a0_ref[...] + a1_ref[...] + y_ref[...]) * dinv + b_ref[...]
    h = jnp.maximum(h, 0.0)
    y2_ref[...] = jnp.dot(h * dinv, w_ref[...],
                          preferred_element_type=jnp.float32)


def _tc_last_body(a0_ref, a1_ref, y_ref, dinv_ref, b_ref, out_ref):
    out_ref[...] = ((a0_ref[...] + a1_ref[...] + y_ref[...]) * dinv_ref[...]
                    + b_ref[...])


_row_spec = pl.BlockSpec((BR, D), lambda i: (i, 0))
_col_spec = pl.BlockSpec((BR, 1), lambda i: (i, 0))
_mat_spec = pl.BlockSpec((D, D), lambda i: (0, 0))
_bias_spec = pl.BlockSpec((1, D), lambda i: (0, 0))
_GRID = (N // BR,)

_tc_first = pl.pallas_call(
    _tc_first_body,
    grid=_GRID,
    in_specs=[_col_spec, _col_spec, _row_spec, _mat_spec],
    out_specs=[_row_spec, _col_spec],
    out_shape=[jax.ShapeDtypeStruct((N, D), jnp.float32),
               jax.ShapeDtypeStruct((N, 1), jnp.float32)],
)

_tc_mid = pl.pallas_call(
    _tc_mid_body,
    grid=_GRID,
    in_specs=[_row_spec, _row_spec, _row_spec, _col_spec, _bias_spec,
              _mat_spec],
    out_specs=_row_spec,
    out_shape=jax.ShapeDtypeStruct((N, D), jnp.float32),
)

_tc_last = pl.pallas_call(
    _tc_last_body,
    grid=_GRID,
    in_specs=[_row_spec, _row_spec, _row_spec, _col_spec, _bias_spec],
    out_specs=_row_spec,
    out_shape=jax.ShapeDtypeStruct((N, D), jnp.float32),
)


def kernel(x, edge_index, W1, b1, W2, b2):
    src = edge_index[0]
    dst = edge_index[1]
    pad = NW * EPT - E
    src_p = jnp.concatenate([src, jnp.zeros((pad,), jnp.int32)])
    dst_p = jnp.concatenate([dst, jnp.full((pad,), N, jnp.int32)])
    src3 = src_p.reshape(NW, NCHUNK, CHUNK)
    dst3 = dst_p.reshape(NW, NCHUNK, CHUNK)

    ones_c = jnp.ones((CHUNK,), jnp.float32)
    zeros_1 = jnp.zeros((ZROWS,), jnp.float32)
    zeros_r = jnp.zeros((ZROWS, D), jnp.float32)

    hist = _sc_degree(dst3, ones_c, zeros_1)
    h0 = hist[:N].reshape(N, 1)
    h1 = hist[ACC_ROWS:ACC_ROWS + N].reshape(N, 1)

    y1, dinv = _tc_first(h0, h1, x, W1)

    acc1 = _sc_scatter(y1, src3, dst3, zeros_r)
    a0 = acc1[:N]
    a1 = acc1[ACC_ROWS:ACC_ROWS + N]

    y2 = _tc_mid(a0, a1, y1, dinv, b1.reshape(1, D), W2)

    acc2 = _sc_scatter(y2, src3, dst3, zeros_r)
    c0 = acc2[:N]
    c1 = acc2[ACC_ROWS:ACC_ROWS + N]

    return _tc_last(c0, c1, y2, dinv, b2.reshape(1, D))


# speedup vs baseline: 21.0635x; 21.0635x over previous
import functools

import jax
import jax.numpy as jnp
from jax import lax
from jax.experimental import pallas as pl
from jax.experimental.pallas import tpu as pltpu
from jax.experimental.pallas import tpu_sc as plsc

N = 10000
D = 128
E = 320000

NC = 2
NS = 16
NW = NC * NS
HALF = 5000
LROWS = 5096
TRASH = HALF
CHUNK = 128
SLICE = 20224
NSL = SLICE // CHUNK
ZR = 312
ZREM = LROWS - NS * ZR

_mesh = plsc.VectorSubcoreMesh(core_axis_name="c", subcore_axis_name="s")
_params = pltpu.CompilerParams(needs_layout_passes=False,
                               use_tc_tiling_on_sc=False)


@functools.partial(
    pl.kernel,
    out_type=[
        jax.ShapeDtypeStruct((NC * LROWS,), jnp.float32),
        jax.ShapeDtypeStruct((NW, NSL, CHUNK), jnp.int32),
        jax.ShapeDtypeStruct((NW, NSL, CHUNK), jnp.int32),
        jax.ShapeDtypeStruct((NW, 16), jnp.int32),
    ],
    mesh=_mesh,
    scratch_types=[
        pltpu.VMEM((SLICE,), jnp.int32),
        pltpu.VMEM((SLICE,), jnp.int32),
        pltpu.VMEM((NSL, CHUNK), jnp.int32),
        pltpu.VMEM((NSL, CHUNK), jnp.int32),
        pltpu.VMEM((CHUNK,), jnp.float32),
        pltpu.VMEM((ZR + ZREM,), jnp.float32),
        pltpu.VMEM((16,), jnp.int32),
        pltpu.VMEM_SHARED((LROWS,), jnp.float32),
    ],
    compiler_params=_params,
)
def _sc_route(src_hbm, dst_hbm, zeros_hbm, ones_hbm, deg_out, srcc_out,
              dstc_out, cnt_out, src_v, dst_v, src_c, dst_c, ones_v, zbuf,
              cnt_v, hist):
    cid = lax.axis_index("c")
    sid = lax.axis_index("s")
    wid = cid * NS + sid
    pltpu.sync_copy(src_hbm.at[sid], src_v)
    pltpu.sync_copy(dst_hbm.at[sid], dst_v)
    pltpu.sync_copy(ones_hbm, ones_v)
    pltpu.sync_copy(zeros_hbm, zbuf)
    pltpu.sync_copy(zbuf.at[pl.ds(0, ZR)], hist.at[pl.ds(sid * ZR, ZR)])

    @pl.when(sid == 0)
    def _():
        pltpu.sync_copy(zbuf.at[pl.ds(0, ZREM)],
                        hist.at[pl.ds(NS * ZR, ZREM)])

    zsrc = jnp.zeros((16,), jnp.int32)
    ztrash = jnp.full((16,), TRASH, jnp.int32)

    def fill(r, c):
        for k in range(CHUNK // 16):
            src_c[r, pl.ds(k * 16, 16)] = zsrc
            dst_c[r, pl.ds(k * 16, 16)] = ztrash
        return c

    lax.fori_loop(0, NSL, fill, 0)

    lo = cid * HALF

    def compact(i, off):
        p = pl.multiple_of(i * 16, 16)
        vs = src_v[pl.ds(p, 16)]
        vd = dst_v[pl.ds(p, 16)]
        ld = vd - lo
        m = (ld >= 0) & (ld < HALF)
        mi = jnp.where(m, 1, 0).astype(jnp.int32)
        pos = off - 1 + plsc.cumsum(mi)
        row = lax.shift_right_arithmetic(pos, 7)
        col = lax.bitwise_and(pos, 127)
        plsc.store_scatter(dst_c, [row, col], ld, mask=m)
        plsc.store_scatter(src_c, [row, col], vs, mask=m)
        return off + jnp.sum(mi)

    cnt = lax.fori_loop(0, SLICE // 16, compact, jnp.int32(0))

    plsc.subcore_barrier()
    nch = (cnt + CHUNK - 1) // CHUNK

    def hbody(j, c):
        pltpu.sync_copy(ones_v, hist.at[dst_c.at[j]], add=True)
        return c

    lax.fori_loop(0, nch, hbody, 0)

    pltpu.sync_copy(src_c, srcc_out.at[wid])
    pltpu.sync_copy(dst_c, dstc_out.at[wid])
    cnt_v[...] = jnp.full((16,), 1, jnp.int32) * cnt
    pltpu.sync_copy(cnt_v, cnt_out.at[wid])

    plsc.subcore_barrier()
    pltpu.sync_copy(hist.at[pl.ds(sid * ZR, ZR)], zbuf.at[pl.ds(0, ZR)])
    pltpu.sync_copy(zbuf.at[pl.ds(0, ZR)],
                    deg_out.at[pl.ds(cid * LROWS + sid * ZR, ZR)])

    @pl.when(sid == 0)
    def _():
        pltpu.sync_copy(hist.at[pl.ds(NS * ZR, ZREM)],
                        zbuf.at[pl.ds(0, ZREM)])
        pltpu.sync_copy(zbuf.at[pl.ds(0, ZREM)],
                        deg_out.at[pl.ds(cid * LROWS + NS * ZR, ZREM)])


@functools.partial(
    pl.kernel,
    out_type=jax.ShapeDtypeStruct((NC * LROWS, D), jnp.float32),
    mesh=_mesh,
    scratch_types=[
        pltpu.VMEM((NSL, CHUNK), jnp.int32),
        pltpu.VMEM((NSL, CHUNK), jnp.int32),
        pltpu.VMEM((16,), jnp.int32),
        pltpu.VMEM((2, CHUNK, D), jnp.float32),
        pltpu.VMEM((CHUNK, D), jnp.float32),
        pltpu.VMEM_SHARED((LROWS, D), jnp.float32),
        pltpu.SemaphoreType.DMA,
        pltpu.SemaphoreType.DMA,
    ],
    compiler_params=_params,
)
def _sc_scatter(y_hbm, srcc_hbm, dstc_hbm, cnt_hbm, zeros_hbm, out_hbm,
                src_v, dst_v, cnt_v, buf, zeros_v, acc, gsem, ssem):
    cid = lax.axis_index("c")
    sid = lax.axis_index("s")
    wid = cid * NS + sid
    pltpu.sync_copy(srcc_hbm.at[wid], src_v)
    pltpu.sync_copy(dstc_hbm.at[wid], dst_v)
    pltpu.sync_copy(cnt_hbm.at[wid], cnt_v)
    cnt = jnp.max(cnt_v[...])
    nch = (cnt + CHUNK - 1) // CHUNK

    @pl.when(nch > 0)
    def _():
        pltpu.async_copy(y_hbm.at[src_v.at[0]], buf.at[0], gsem)

    pltpu.sync_copy(zeros_hbm, zeros_v)
    base = sid * ZR
    pltpu.sync_copy(zeros_v, acc.at[pl.ds(base, CHUNK)])
    pltpu.sync_copy(zeros_v, acc.at[pl.ds(base + CHUNK, CHUNK)])
    pltpu.sync_copy(zeros_v.at[pl.ds(0, ZR - 2 * CHUNK)],
                    acc.at[pl.ds(base + 2 * CHUNK, ZR - 2 * CHUNK)])

    @pl.when(sid == 0)
    def _():
        pltpu.sync_copy(zeros_v.at[pl.ds(0, ZREM)],
                        acc.at[pl.ds(NS * ZR, ZREM)])

    plsc.subcore_barrier()

    def body(j, c):
        @pl.when(j >= 1)
        def _():
            pltpu.make_async_copy(buf.at[(j - 1) % 2],
                                  acc.at[dst_v.at[j - 1]], ssem).wait()

        @pl.when(j + 1 < nch)
        def _():
            pltpu.async_copy(y_hbm.at[src_v.at[j + 1]], buf.at[(j + 1) % 2],
                             gsem)

        pltpu.make_async_copy(y_hbm.at[src_v.at[j]], buf.at[j % 2],
                              gsem).wait()
        pltpu.async_copy(buf.at[j % 2], acc.at[dst_v.at[j]], ssem, add=True)
        return c

    lax.fori_loop(0, nch, body, 0)

    @pl.when(nch > 0)
    def _():
        pltpu.make_async_copy(buf.at[(nch - 1) % 2],
                              acc.at[dst_v.at[nch - 1]], ssem).wait()

    plsc.subcore_barrier()

    obase = cid * LROWS + base
    hbm_pend = []
    for k in range(3):
        rows = CHUNK if k < 2 else ZR - 2 * CHUNK
        b = buf.at[k % 2] if rows == CHUNK else buf.at[k % 2].at[pl.ds(0, rows)]
        pltpu.sync_copy(acc.at[pl.ds(base + k * CHUNK, rows)], b)
        hbm_pend.append(
            pltpu.async_copy(b, out_hbm.at[pl.ds(obase + k * CHUNK, rows)],
                             gsem))
        if k >= 1:
            hbm_pend[k - 1].wait()
    hbm_pend[2].wait()

    @pl.when(sid == 0)
    def _():
        b = buf.at[0].at[pl.ds(0, ZREM)]
        pltpu.sync_copy(acc.at[pl.ds(NS * ZR, ZREM)], b)
        pltpu.sync_copy(b, out_hbm.at[pl.ds(cid * LROWS + NS * ZR, ZREM)])


BR = 1000
NB = N // BR


def _tc_matmul_body(x_ref, w_ref, xw_ref):
    xw_ref[...] = jnp.dot(x_ref[...], w_ref[...],
                          preferred_element_type=jnp.float32)


def _tc_scale_body(h_ref, xw_ref, y_ref, dinv_ref):
    dinv = lax.rsqrt(h_ref[...] + 1.0)
    y_ref[...] = xw_ref[...] * dinv
    dinv_ref[...] = dinv


def _tc_mid_body(a_ref, y_ref, dinv_ref, b_ref, w_ref, y2_ref):
    dinv = dinv_ref[...]
    h = jnp.maximum((a_ref[...] + y_ref[...]) * dinv + b_ref[...], 0.0)
    y2_ref[...] = jnp.dot(h * dinv, w_ref[...],
                          preferred_element_type=jnp.float32)


def _tc_last_body(a_ref, y_ref, dinv_ref, b_ref, out_ref):
    out_ref[...] = (a_ref[...] + y_ref[...]) * dinv_ref[...] + b_ref[...]


_row = pl.BlockSpec((BR, D), lambda i: (i, 0))
_col = pl.BlockSpec((BR, 1), lambda i: (i, 0))
_mat = pl.BlockSpec((D, D), lambda i: (0, 0))
_bias = pl.BlockSpec((1, D), lambda i: (0, 0))

_tc_matmul = pl.pallas_call(
    _tc_matmul_body,
    grid=(NB,),
    in_specs=[_row, _mat],
    out_specs=_row,
    out_shape=jax.ShapeDtypeStruct((N, D), jnp.float32),
)

_tc_scale = pl.pallas_call(
    _tc_scale_body,
    grid=(NB,),
    in_specs=[_col, _row],
    out_specs=[_row, _col],
    out_shape=[jax.ShapeDtypeStruct((N, D), jnp.float32),
               jax.ShapeDtypeStruct((N, 1), jnp.float32)],
)

_tc_mid = pl.pallas_call(
    _tc_mid_body,
    grid=(NB,),
    in_specs=[_row, _row, _col, _bias, _mat],
    out_specs=_row,
    out_shape=jax.ShapeDtypeStruct((N, D), jnp.float32),
)

_tc_last = pl.pallas_call(
    _tc_last_body,
    grid=(NB,),
    in_specs=[_row, _row, _col, _bias],
    out_specs=_row,
    out_shape=jax.ShapeDtypeStruct((N, D), jnp.float32),
)


def _merge(a):
    return jnp.concatenate([a[:HALF], a[LROWS:LROWS + HALF]], axis=0)


def kernel(x, edge_index, W1, b1, W2, b2):
    src = edge_index[0]
    dst = edge_index[1]
    pad = NS * SLICE - E
    src_p = jnp.concatenate([src, jnp.zeros((pad,), jnp.int32)])
    dst_p = jnp.concatenate([dst, jnp.full((pad,), N, jnp.int32)])
    src2 = src_p.reshape(NS, SLICE)
    dst2 = dst_p.reshape(NS, SLICE)

    ones_c = jnp.ones((CHUNK,), jnp.float32)
    zeros_z = jnp.zeros((ZR + ZREM,), jnp.float32)
    zeros_r = jnp.zeros((CHUNK, D), jnp.float32)

    deg, srcc, dstc, cnts = _sc_route(src2, dst2, zeros_z, ones_c)
    xw1 = _tc_matmul(x, W1)

    h_col = _merge(deg).reshape(N, 1)
    y1, dinv = _tc_scale(h_col, xw1)

    acc1 = _merge(_sc_scatter(y1, srcc, dstc, cnts, zeros_r))
    y2 = _tc_mid(acc1, y1, dinv, b1.reshape(1, D), W2)
    acc2 = _merge(_sc_scatter(y2, srcc, dstc, cnts, zeros_r))
    return _tc_last(acc2, y2, dinv, b2.reshape(1, D))
